# baseline (speedup 1.0000x reference)
import functools

import jax
import jax.numpy as jnp
from jax import lax
from jax.experimental import pallas as pl
from jax.experimental.pallas import tpu as pltpu
from jax.experimental.pallas import tpu_sc as plsc

N = 10000
F = 128
H = 64
E = 320000
NC = 2
NS = 16
NP = 10240
RPT = NP // NS
EPT = E // (NC * NS)
K = 128
NB = E // K
BPT = NB // (NC * NS)
NBUF = 3
RCH = RPT // K
NH = NP // 2

_mesh = plsc.VectorSubcoreMesh(core_axis_name="c", subcore_axis_name="s")


def _mdst_transform(idx_ref, j):
    for c in range(K // 16):
        d = idx_ref[j, 1, pl.ds(c * 16, 16)]
        m = lax.shift_right_logical(d, 1) + jnp.bitwise_and(d, 1) * NH
        idx_ref[j, 1, pl.ds(c * 16, 16)] = m


@functools.partial(
    pl.kernel,
    mesh=_mesh,
    out_type=jax.ShapeDtypeStruct((NC * NP,), jnp.float32),
    compiler_params=pltpu.CompilerParams(use_tc_tiling_on_sc=False),
    scratch_types=[
        pltpu.VMEM((BPT, 2, K), jnp.int32),
        pltpu.VMEM((2, K), jnp.int32),
        pltpu.VMEM((K,), jnp.float32),
        pltpu.VMEM((RPT,), jnp.float32),
        pltpu.VMEM_SHARED((NP,), jnp.float32),
    ],
)
def _deg_kernel(e3_hbm, zeros_hbm, ones_hbm, out_hbm, idx_v, ex_v, ones_v, wb_v, acc_sh):
    cid = lax.axis_index("c")
    sid = lax.axis_index("s")
    w = cid * NS + sid
    pltpu.sync_copy(ones_hbm, ones_v)
    pltpu.sync_copy(zeros_hbm, wb_v)
    pltpu.sync_copy(wb_v, acc_sh.at[pl.ds(sid * RPT, RPT)])
    pltpu.sync_copy(e3_hbm.at[pl.ds(w * BPT, BPT)], idx_v)

    @pl.when(w < NB - NC * NS * BPT)
    def _():
        pltpu.sync_copy(e3_hbm.at[NC * NS * BPT + w], ex_v)

    plsc.subcore_barrier()

    def body(j, carry):
        _mdst_transform(idx_v, j)
        pltpu.sync_copy(ones_v, acc_sh.at[idx_v.at[j, 1]], add=True)
        return carry

    lax.fori_loop(0, BPT, body, 0)

    @pl.when(w < NB - NC * NS * BPT)
    def _():
        for c in range(K // 16):
            d = ex_v[1, pl.ds(c * 16, 16)]
            m = lax.shift_right_logical(d, 1) + jnp.bitwise_and(d, 1) * NH
            ex_v[1, pl.ds(c * 16, 16)] = m
        pltpu.sync_copy(ones_v, acc_sh.at[ex_v.at[1]], add=True)

    plsc.subcore_barrier()
    pltpu.sync_copy(acc_sh.at[pl.ds(sid * RPT, RPT)], wb_v)
    pltpu.sync_copy(wb_v, out_hbm.at[pl.ds(cid * NP + sid * RPT, RPT)])


@functools.partial(
    pl.kernel,
    mesh=_mesh,
    out_type=[
        jax.ShapeDtypeStruct((NP, H), jnp.float32),
        jax.ShapeDtypeStruct((NP, H), jnp.float32),
    ],
    compiler_params=pltpu.CompilerParams(use_tc_tiling_on_sc=False),
    scratch_types=[
        pltpu.VMEM((BPT, 2, K), jnp.int32),
        pltpu.VMEM((2, K), jnp.int32),
        [pltpu.VMEM((K, H), jnp.float32)] * NBUF,
        pltpu.VMEM_SHARED((NP, H), jnp.float32),
        [pltpu.SemaphoreType.DMA] * NBUF,
    ],
)
def _agg_kernel(g_hbm, e3_hbm, zeros_hbm, out0_hbm, out1_hbm,
                idx_v, ex_v, rows_v, acc_sh, sems):
    cid = lax.axis_index("c")
    sid = lax.axis_index("s")
    w = cid * NS + sid
    pltpu.sync_copy(zeros_hbm, rows_v[0])
    for r in range(RCH):
        pltpu.sync_copy(rows_v[0], acc_sh.at[pl.ds(sid * RPT + r * K, K)])
    pltpu.sync_copy(e3_hbm.at[pl.ds(w * BPT, BPT)], idx_v)

    @pl.when(w < NB - NC * NS * BPT)
    def _():
        pltpu.sync_copy(e3_hbm.at[NC * NS * BPT + w], ex_v)

    plsc.subcore_barrier()

    for b in range(NBUF):
        pltpu.async_copy(g_hbm.at[idx_v.at[b, 0]], rows_v[b], sems[b])

    def body(i, carry):
        j0 = NBUF * i
        for b in range(NBUF):
            j = j0 + b
            pltpu.make_async_copy(g_hbm.at[idx_v.at[j, 0]], rows_v[b], sems[b]).wait()
            pltpu.async_copy(rows_v[b], acc_sh.at[idx_v.at[j, 1]], sems[b], add=True)
            bp = (b - 1) % NBUF

            @pl.when(j > 0)
            def _():
                pltpu.make_async_copy(
                    rows_v[bp], acc_sh.at[idx_v.at[j - 1, 1]], sems[bp]).wait()

                @pl.when(j - 1 + NBUF < BPT)
                def _():
                    pltpu.async_copy(
                        g_hbm.at[idx_v.at[j - 1 + NBUF, 0]], rows_v[bp], sems[bp])

        return carry

    lax.fori_loop(0, BPT // NBUF, body, 0)
    lastb = (BPT - 1) % NBUF
    pltpu.make_async_copy(
        rows_v[lastb], acc_sh.at[idx_v.at[BPT - 1, 1]], sems[lastb]).wait()

    @pl.when(w < NB - NC * NS * BPT)
    def _():
        pltpu.sync_copy(g_hbm.at[ex_v.at[0]], rows_v[0])
        pltpu.sync_copy(rows_v[0], acc_sh.at[ex_v.at[1]], add=True)

    plsc.subcore_barrier()
    for r in range(RCH):
        b = r % NBUF
        pltpu.sync_copy(acc_sh.at[pl.ds(sid * RPT + r * K, K)], rows_v[b])

        @pl.when(cid == 0)
        def _():
            pltpu.sync_copy(rows_v[b], out0_hbm.at[pl.ds(sid * RPT + r * K, K)])

        @pl.when(cid == 1)
        def _():
            pltpu.sync_copy(rows_v[b], out1_hbm.at[pl.ds(sid * RPT + r * K, K)])


def _dinv_packed(deg_ref):
    de = lax.rsqrt(deg_ref[0:NH] + deg_ref[NP:NP + NH] + 1.0)
    do = lax.rsqrt(deg_ref[NH:NP] + deg_ref[NP + NH:2 * NP] + 1.0)
    return jnp.concatenate(
        [jnp.broadcast_to(jnp.reshape(de, (NH, 1)), (NH, H)),
         jnp.broadcast_to(jnp.reshape(do, (NH, 1)), (NH, H))], axis=1)


def _tc_first(deg_ref, xp_ref, w1d_ref, g_ref):
    dp = _dinv_packed(deg_ref)
    h = jnp.dot(xp_ref[...], w1d_ref[...], preferred_element_type=jnp.float32)
    g_ref[0:N // 2] = h * dp[0:N // 2]
    g_ref[N // 2:NH] = jnp.zeros((NH - N // 2, 2 * H), jnp.float32)


def _tc_mid(deg_ref, p0_ref, p1_ref, g_ref, b_ref, w2d_ref, g2_ref):
    dp = _dinv_packed(deg_ref)
    z = dp * (p0_ref[...] + p1_ref[...] + g_ref[...]) + b_ref[...]
    z = jnp.maximum(z, 0.0)
    g2_ref[...] = jnp.dot(z, w2d_ref[...], preferred_element_type=jnp.float32) * dp


def _tc_last(deg_ref, p0_ref, p1_ref, g_ref, b_ref, out_ref):
    z = _dinv_packed(deg_ref) * (p0_ref[...] + p1_ref[...] + g_ref[...]) + b_ref[...]
    out_ref[...] = jnp.maximum(z, 0.0)


def kernel(x, edge_index, W1, b1, W2, b2):
    ei = edge_index.astype(jnp.int32)
    e3 = ei.reshape(2, NB, K).transpose(1, 0, 2)

    zeros_row = jnp.zeros((RPT,), jnp.float32)
    ones_row = jnp.ones((K,), jnp.float32)
    zeros_blk = jnp.zeros((K, H), jnp.float32)

    deg = _deg_kernel(e3, zeros_row, ones_row)

    xp = x.reshape(N // 2, 2 * F)
    zf = jnp.zeros((F, H), jnp.float32)
    zh = jnp.zeros((H, H), jnp.float32)
    w1d = jnp.concatenate(
        [jnp.concatenate([W1, zf], axis=1), jnp.concatenate([zf, W1], axis=1)],
        axis=0)
    w2d = jnp.concatenate(
        [jnp.concatenate([W2, zh], axis=1), jnp.concatenate([zh, W2], axis=1)],
        axis=0)
    b1r = jnp.concatenate([b1, b1]).reshape(1, 2 * H)
    b2r = jnp.concatenate([b2, b2]).reshape(1, 2 * H)

    g1p = pl.pallas_call(
        _tc_first,
        out_shape=jax.ShapeDtypeStruct((NH, 2 * H), jnp.float32),
    )(deg, xp, w1d)

    p10, p11 = _agg_kernel(g1p.reshape(NP, H), e3, zeros_blk)

    g2p = pl.pallas_call(
        _tc_mid,
        out_shape=jax.ShapeDtypeStruct((NH, 2 * H), jnp.float32),
    )(deg, p10.reshape(NH, 2 * H), p11.reshape(NH, 2 * H), g1p, b1r, w2d)

    p20, p21 = _agg_kernel(g2p.reshape(NP, H), e3, zeros_blk)

    outp = pl.pallas_call(
        _tc_last,
        out_shape=jax.ShapeDtypeStruct((NH, 2 * H), jnp.float32),
    )(deg, p20.reshape(NH, 2 * H), p21.reshape(NH, 2 * H), g2p, b2r)

    return outp.reshape(NP, H)[:N]


# speedup vs baseline: 1.0364x; 1.0364x over previous
import functools

import jax
import jax.numpy as jnp
from jax import lax
from jax.experimental import pallas as pl
from jax.experimental.pallas import tpu as pltpu
from jax.experimental.pallas import tpu_sc as plsc

N = 10000
F = 128
H = 64
E = 320000
NC = 2
NS = 16
NP = 10240
RPT = NP // NS
EPT = E // (NC * NS)
K = 128
NB = E // K
BPT = NB // (NC * NS)
NBUF = 3
RCH = RPT // K
NH = NP // 2

_mesh = plsc.VectorSubcoreMesh(core_axis_name="c", subcore_axis_name="s")


def _mdst_transform(idx_ref, j):
    for c in range(K // 16):
        d = idx_ref[j, 1, pl.ds(c * 16, 16)]
        m = lax.shift_right_logical(d, 1) + jnp.bitwise_and(d, 1) * NH
        idx_ref[j, 1, pl.ds(c * 16, 16)] = m


@functools.partial(
    pl.kernel,
    mesh=_mesh,
    out_type=jax.ShapeDtypeStruct((NC * NP,), jnp.float32),
    compiler_params=pltpu.CompilerParams(use_tc_tiling_on_sc=False),
    scratch_types=[
        pltpu.VMEM((BPT, 2, K), jnp.int32),
        pltpu.VMEM((2, K), jnp.int32),
        pltpu.VMEM((K,), jnp.float32),
        pltpu.VMEM((RPT,), jnp.float32),
        pltpu.VMEM_SHARED((NP,), jnp.float32),
    ],
)
def _deg_kernel(e3_hbm, zeros_hbm, ones_hbm, out_hbm, idx_v, ex_v, ones_v, wb_v, acc_sh):
    cid = lax.axis_index("c")
    sid = lax.axis_index("s")
    w = cid * NS + sid
    pltpu.sync_copy(ones_hbm, ones_v)
    pltpu.sync_copy(zeros_hbm, wb_v)
    pltpu.sync_copy(wb_v, acc_sh.at[pl.ds(sid * RPT, RPT)])
    pltpu.sync_copy(e3_hbm.at[pl.ds(w * BPT, BPT)], idx_v)

    @pl.when(w < NB - NC * NS * BPT)
    def _():
        pltpu.sync_copy(e3_hbm.at[NC * NS * BPT + w], ex_v)

    plsc.subcore_barrier()

    def body(j, carry):
        _mdst_transform(idx_v, j)
        pltpu.sync_copy(ones_v, acc_sh.at[idx_v.at[j, 1]], add=True)
        return carry

    lax.fori_loop(0, BPT, body, 0)

    @pl.when(w < NB - NC * NS * BPT)
    def _():
        for c in range(K // 16):
            d = ex_v[1, pl.ds(c * 16, 16)]
            m = lax.shift_right_logical(d, 1) + jnp.bitwise_and(d, 1) * NH
            ex_v[1, pl.ds(c * 16, 16)] = m
        pltpu.sync_copy(ones_v, acc_sh.at[ex_v.at[1]], add=True)

    plsc.subcore_barrier()
    pltpu.sync_copy(acc_sh.at[pl.ds(sid * RPT, RPT)], wb_v)
    pltpu.sync_copy(wb_v, out_hbm.at[pl.ds(cid * NP + sid * RPT, RPT)])


@functools.partial(
    pl.kernel,
    mesh=_mesh,
    out_type=[
        jax.ShapeDtypeStruct((NP, H), jnp.float32),
        jax.ShapeDtypeStruct((NP, H), jnp.float32),
    ],
    compiler_params=pltpu.CompilerParams(use_tc_tiling_on_sc=False),
    scratch_types=[
        pltpu.VMEM((BPT, 2, K), jnp.int32),
        pltpu.VMEM((2, K), jnp.int32),
        [pltpu.VMEM((K, H), jnp.float32)] * NBUF,
        pltpu.VMEM_SHARED((NP, H), jnp.float32),
        [pltpu.SemaphoreType.DMA] * NBUF,
    ],
)
def _agg_kernel(g_hbm, e3_hbm, zeros_hbm, out0_hbm, out1_hbm,
                idx_v, ex_v, rows_v, acc_sh, sems):
    cid = lax.axis_index("c")
    sid = lax.axis_index("s")
    w = cid * NS + sid
    pltpu.sync_copy(zeros_hbm, rows_v[0])
    for r in range(RCH):
        pltpu.sync_copy(rows_v[0], acc_sh.at[pl.ds(sid * RPT + r * K, K)])
    pltpu.sync_copy(e3_hbm.at[pl.ds(w * BPT, BPT)], idx_v)

    @pl.when(w < NB - NC * NS * BPT)
    def _():
        pltpu.sync_copy(e3_hbm.at[NC * NS * BPT + w], ex_v)

    plsc.subcore_barrier()

    for b in range(NBUF):
        pltpu.async_copy(g_hbm.at[idx_v.at[b, 0]], rows_v[b], sems[b])

    def body(i, carry):
        j0 = NBUF * i
        for b in range(NBUF):
            j = j0 + b
            pltpu.make_async_copy(g_hbm.at[idx_v.at[j, 0]], rows_v[b], sems[b]).wait()
            pltpu.sync_copy(rows_v[b], acc_sh.at[idx_v.at[j, 1]], add=True)

            @pl.when(j + NBUF < BPT)
            def _():
                pltpu.async_copy(g_hbm.at[idx_v.at[j + NBUF, 0]], rows_v[b], sems[b])

        return carry

    lax.fori_loop(0, BPT // NBUF, body, 0)

    @pl.when(w < NB - NC * NS * BPT)
    def _():
        pltpu.sync_copy(g_hbm.at[ex_v.at[0]], rows_v[0])
        pltpu.sync_copy(rows_v[0], acc_sh.at[ex_v.at[1]], add=True)

    plsc.subcore_barrier()
    for r in range(RCH):
        b = r % NBUF
        pltpu.sync_copy(acc_sh.at[pl.ds(sid * RPT + r * K, K)], rows_v[b])

        @pl.when(cid == 0)
        def _():
            pltpu.sync_copy(rows_v[b], out0_hbm.at[pl.ds(sid * RPT + r * K, K)])

        @pl.when(cid == 1)
        def _():
            pltpu.sync_copy(rows_v[b], out1_hbm.at[pl.ds(sid * RPT + r * K, K)])


def _dinv_packed(deg_ref):
    de = lax.rsqrt(deg_ref[0:NH] + deg_ref[NP:NP + NH] + 1.0)
    do = lax.rsqrt(deg_ref[NH:NP] + deg_ref[NP + NH:2 * NP] + 1.0)
    return jnp.concatenate(
        [jnp.broadcast_to(jnp.reshape(de, (NH, 1)), (NH, H)),
         jnp.broadcast_to(jnp.reshape(do, (NH, 1)), (NH, H))], axis=1)


def _tc_first(deg_ref, xp_ref, w1d_ref, g_ref):
    dp = _dinv_packed(deg_ref)
    h = jnp.dot(xp_ref[...], w1d_ref[...], preferred_element_type=jnp.float32)
    g_ref[0:N // 2] = h * dp[0:N // 2]
    g_ref[N // 2:NH] = jnp.zeros((NH - N // 2, 2 * H), jnp.float32)


def _tc_mid(deg_ref, p0_ref, p1_ref, g_ref, b_ref, w2d_ref, g2_ref):
    dp = _dinv_packed(deg_ref)
    z = dp * (p0_ref[...] + p1_ref[...] + g_ref[...]) + b_ref[...]
    z = jnp.maximum(z, 0.0)
    g2_ref[...] = jnp.dot(z, w2d_ref[...], preferred_element_type=jnp.float32) * dp


def _tc_last(deg_ref, p0_ref, p1_ref, g_ref, b_ref, out_ref):
    z = _dinv_packed(deg_ref) * (p0_ref[...] + p1_ref[...] + g_ref[...]) + b_ref[...]
    out_ref[...] = jnp.maximum(z, 0.0)


def kernel(x, edge_index, W1, b1, W2, b2):
    ei = edge_index.astype(jnp.int32)
    e3 = ei.reshape(2, NB, K).transpose(1, 0, 2)

    zeros_row = jnp.zeros((RPT,), jnp.float32)
    ones_row = jnp.ones((K,), jnp.float32)
    zeros_blk = jnp.zeros((K, H), jnp.float32)

    deg = _deg_kernel(e3, zeros_row, ones_row)

    xp = x.reshape(N // 2, 2 * F)
    zf = jnp.zeros((F, H), jnp.float32)
    zh = jnp.zeros((H, H), jnp.float32)
    w1d = jnp.concatenate(
        [jnp.concatenate([W1, zf], axis=1), jnp.concatenate([zf, W1], axis=1)],
        axis=0)
    w2d = jnp.concatenate(
        [jnp.concatenate([W2, zh], axis=1), jnp.concatenate([zh, W2], axis=1)],
        axis=0)
    b1r = jnp.concatenate([b1, b1]).reshape(1, 2 * H)
    b2r = jnp.concatenate([b2, b2]).reshape(1, 2 * H)

    g1p = pl.pallas_call(
        _tc_first,
        out_shape=jax.ShapeDtypeStruct((NH, 2 * H), jnp.float32),
    )(deg, xp, w1d)

    p10, p11 = _agg_kernel(g1p.reshape(NP, H), e3, zeros_blk)

    g2p = pl.pallas_call(
        _tc_mid,
        out_shape=jax.ShapeDtypeStruct((NH, 2 * H), jnp.float32),
    )(deg, p10.reshape(NH, 2 * H), p11.reshape(NH, 2 * H), g1p, b1r, w2d)

    p20, p21 = _agg_kernel(g2p.reshape(NP, H), e3, zeros_blk)

    outp = pl.pallas_call(
        _tc_last,
        out_shape=jax.ShapeDtypeStruct((NH, 2 * H), jnp.float32),
    )(deg, p20.reshape(NH, 2 * H), p21.reshape(NH, 2 * H), g2p, b2r)

    return outp.reshape(NP, H)[:N]


# speedup vs baseline: 1.0528x; 1.0158x over previous
import functools

import jax
import jax.numpy as jnp
from jax import lax
from jax.experimental import pallas as pl
from jax.experimental.pallas import tpu as pltpu
from jax.experimental.pallas import tpu_sc as plsc

N = 10000
F = 128
H = 64
E = 320000
NC = 2
NS = 16
NP = 10240
RPT = NP // NS
EPT = E // (NC * NS)
K = 128
NB = E // K
BPT = NB // (NC * NS)
NBUF = 3
RCH = RPT // K
NH = NP // 2

_mesh = plsc.VectorSubcoreMesh(core_axis_name="c", subcore_axis_name="s")


def _mdst_transform(idx_ref, j):
    for c in range(K // 16):
        d = idx_ref[j, 1, pl.ds(c * 16, 16)]
        m = lax.shift_right_logical(d, 1) + jnp.bitwise_and(d, 1) * NH
        idx_ref[j, 1, pl.ds(c * 16, 16)] = m


@functools.partial(
    pl.kernel,
    mesh=_mesh,
    out_type=jax.ShapeDtypeStruct((NC * NP,), jnp.float32),
    compiler_params=pltpu.CompilerParams(use_tc_tiling_on_sc=False),
    scratch_types=[
        pltpu.VMEM((BPT, 2, K), jnp.int32),
        pltpu.VMEM((2, K), jnp.int32),
        pltpu.VMEM((K,), jnp.float32),
        pltpu.VMEM((RPT,), jnp.float32),
        pltpu.VMEM_SHARED((NP,), jnp.float32),
        pltpu.SemaphoreType.DMA,
    ],
)
def _deg_kernel(e3_hbm, zeros_hbm, ones_hbm, out_hbm, idx_v, ex_v, ones_v, wb_v,
                acc_sh, sem):
    cid = lax.axis_index("c")
    sid = lax.axis_index("s")
    w = cid * NS + sid
    pltpu.sync_copy(ones_hbm, ones_v)
    pltpu.sync_copy(zeros_hbm, wb_v)
    pltpu.sync_copy(wb_v, acc_sh.at[pl.ds(sid * RPT, RPT)])
    pltpu.sync_copy(e3_hbm.at[pl.ds(w * BPT, BPT)], idx_v)

    @pl.when(w < NB - NC * NS * BPT)
    def _():
        pltpu.sync_copy(e3_hbm.at[NC * NS * BPT + w], ex_v)

    plsc.subcore_barrier()

    def body(j, carry):
        _mdst_transform(idx_v, j)
        pltpu.async_copy(ones_v, acc_sh.at[idx_v.at[j, 1]], sem, add=True)

        @pl.when(j > 0)
        def _():
            pltpu.make_async_copy(ones_v, acc_sh.at[idx_v.at[0, 1]], sem).wait()

        return carry

    lax.fori_loop(0, BPT, body, 0)
    pltpu.make_async_copy(ones_v, acc_sh.at[idx_v.at[0, 1]], sem).wait()

    @pl.when(w < NB - NC * NS * BPT)
    def _():
        for c in range(K // 16):
            d = ex_v[1, pl.ds(c * 16, 16)]
            m = lax.shift_right_logical(d, 1) + jnp.bitwise_and(d, 1) * NH
            ex_v[1, pl.ds(c * 16, 16)] = m
        pltpu.sync_copy(ones_v, acc_sh.at[ex_v.at[1]], add=True)

    plsc.subcore_barrier()
    pltpu.sync_copy(acc_sh.at[pl.ds(sid * RPT, RPT)], wb_v)
    pltpu.sync_copy(wb_v, out_hbm.at[pl.ds(cid * NP + sid * RPT, RPT)])


@functools.partial(
    pl.kernel,
    mesh=_mesh,
    out_type=[
        jax.ShapeDtypeStruct((NP, H), jnp.float32),
        jax.ShapeDtypeStruct((NP, H), jnp.float32),
    ],
    compiler_params=pltpu.CompilerParams(use_tc_tiling_on_sc=False),
    scratch_types=[
        pltpu.VMEM((BPT, 2, K), jnp.int32),
        pltpu.VMEM((2, K), jnp.int32),
        [pltpu.VMEM((K, H), jnp.float32)] * NBUF,
        pltpu.VMEM_SHARED((NP, H), jnp.float32),
        [pltpu.SemaphoreType.DMA] * NBUF,
    ],
)
def _agg_kernel(g_hbm, e3_hbm, zeros_hbm, out0_hbm, out1_hbm,
                idx_v, ex_v, rows_v, acc_sh, sems):
    cid = lax.axis_index("c")
    sid = lax.axis_index("s")
    w = cid * NS + sid
    pltpu.sync_copy(zeros_hbm, rows_v[0])
    for r in range(RCH):
        pltpu.sync_copy(rows_v[0], acc_sh.at[pl.ds(sid * RPT + r * K, K)])
    pltpu.sync_copy(e3_hbm.at[pl.ds(w * BPT, BPT)], idx_v)

    @pl.when(w < NB - NC * NS * BPT)
    def _():
        pltpu.sync_copy(e3_hbm.at[NC * NS * BPT + w], ex_v)

    plsc.subcore_barrier()

    for b in range(NBUF):
        pltpu.async_copy(g_hbm.at[idx_v.at[b, 0]], rows_v[b], sems[b])

    def body(i, carry):
        j0 = NBUF * i
        for b in range(NBUF):
            j = j0 + b
            pltpu.make_async_copy(g_hbm.at[idx_v.at[j, 0]], rows_v[b], sems[b]).wait()
            pltpu.sync_copy(rows_v[b], acc_sh.at[idx_v.at[j, 1]], add=True)

            @pl.when(j + NBUF < BPT)
            def _():
                pltpu.async_copy(g_hbm.at[idx_v.at[j + NBUF, 0]], rows_v[b], sems[b])

        return carry

    lax.fori_loop(0, BPT // NBUF, body, 0)

    @pl.when(w < NB - NC * NS * BPT)
    def _():
        pltpu.sync_copy(g_hbm.at[ex_v.at[0]], rows_v[0])
        pltpu.sync_copy(rows_v[0], acc_sh.at[ex_v.at[1]], add=True)

    plsc.subcore_barrier()
    for r in range(RCH):
        b = r % NBUF
        pltpu.sync_copy(acc_sh.at[pl.ds(sid * RPT + r * K, K)], rows_v[b])

        @pl.when(cid == 0)
        def _():
            pltpu.sync_copy(rows_v[b], out0_hbm.at[pl.ds(sid * RPT + r * K, K)])

        @pl.when(cid == 1)
        def _():
            pltpu.sync_copy(rows_v[b], out1_hbm.at[pl.ds(sid * RPT + r * K, K)])


def _dinv_packed(deg_ref):
    de = lax.rsqrt(deg_ref[0:NH] + deg_ref[NP:NP + NH] + 1.0)
    do = lax.rsqrt(deg_ref[NH:NP] + deg_ref[NP + NH:2 * NP] + 1.0)
    return jnp.concatenate(
        [jnp.broadcast_to(jnp.reshape(de, (NH, 1)), (NH, H)),
         jnp.broadcast_to(jnp.reshape(do, (NH, 1)), (NH, H))], axis=1)


def _tc_first(deg_ref, xp_ref, w1d_ref, g_ref):
    dp = _dinv_packed(deg_ref)
    h = jnp.dot(xp_ref[...], w1d_ref[...], preferred_element_type=jnp.float32)
    g_ref[0:N // 2] = h * dp[0:N // 2]
    g_ref[N // 2:NH] = jnp.zeros((NH - N // 2, 2 * H), jnp.float32)


def _tc_mid(deg_ref, p0_ref, p1_ref, g_ref, b_ref, w2d_ref, g2_ref):
    dp = _dinv_packed(deg_ref)
    z = dp * (p0_ref[...] + p1_ref[...] + g_ref[...]) + b_ref[...]
    z = jnp.maximum(z, 0.0)
    g2_ref[...] = jnp.dot(z, w2d_ref[...], preferred_element_type=jnp.float32) * dp


def _tc_last(deg_ref, p0_ref, p1_ref, g_ref, b_ref, out_ref):
    z = _dinv_packed(deg_ref) * (p0_ref[...] + p1_ref[...] + g_ref[...]) + b_ref[...]
    out_ref[...] = jnp.maximum(z, 0.0)


def kernel(x, edge_index, W1, b1, W2, b2):
    ei = edge_index.astype(jnp.int32)
    e3 = ei.reshape(2, NB, K).transpose(1, 0, 2)

    zeros_row = jnp.zeros((RPT,), jnp.float32)
    ones_row = jnp.ones((K,), jnp.float32)
    zeros_blk = jnp.zeros((K, H), jnp.float32)

    deg = _deg_kernel(e3, zeros_row, ones_row)

    xp = x.reshape(N // 2, 2 * F)
    zf = jnp.zeros((F, H), jnp.float32)
    zh = jnp.zeros((H, H), jnp.float32)
    w1d = jnp.concatenate(
        [jnp.concatenate([W1, zf], axis=1), jnp.concatenate([zf, W1], axis=1)],
        axis=0)
    w2d = jnp.concatenate(
        [jnp.concatenate([W2, zh], axis=1), jnp.concatenate([zh, W2], axis=1)],
        axis=0)
    b1r = jnp.concatenate([b1, b1]).reshape(1, 2 * H)
    b2r = jnp.concatenate([b2, b2]).reshape(1, 2 * H)

    g1p = pl.pallas_call(
        _tc_first,
        out_shape=jax.ShapeDtypeStruct((NH, 2 * H), jnp.float32),
    )(deg, xp, w1d)

    p10, p11 = _agg_kernel(g1p.reshape(NP, H), e3, zeros_blk)

    g2p = pl.pallas_call(
        _tc_mid,
        out_shape=jax.ShapeDtypeStruct((NH, 2 * H), jnp.float32),
    )(deg, p10.reshape(NH, 2 * H), p11.reshape(NH, 2 * H), g1p, b1r, w2d)

    p20, p21 = _agg_kernel(g2p.reshape(NP, H), e3, zeros_blk)

    outp = pl.pallas_call(
        _tc_last,
        out_shape=jax.ShapeDtypeStruct((NH, 2 * H), jnp.float32),
    )(deg, p20.reshape(NH, 2 * H), p21.reshape(NH, 2 * H), g2p, b2r)

    return outp.reshape(NP, H)[:N]
